# baseline (speedup 1.0000x reference)
import functools

import jax
import jax.numpy as jnp
from jax import lax
from jax.experimental import pallas as pl
from jax.experimental.pallas import tpu as pltpu
from jax.experimental.pallas import tpu_sc as plsc

DIM = 64


def _transh_sc(h_ids, r_ids, t_ids, entity_emb, relation_emb, normal_vec):
    B = h_ids.shape[0]
    NC, NS, L = 2, 16, 16
    NW = NC * NS
    PW = B // NW
    C = min(256, PW)
    NCH = PW // C
    IH = min(128, C)
    KV = DIM // L

    mesh = plsc.VectorSubcoreMesh(
        core_axis_name="c", subcore_axis_name="s", num_cores=NC, num_subcores=NS)

    @functools.partial(
        pl.kernel,
        mesh=mesh,
        out_type=jax.ShapeDtypeStruct((B,), jnp.float32),
        compiler_params=pltpu.CompilerParams(
            needs_layout_passes=False, use_tc_tiling_on_sc=False),
        scratch_types=[
            [pltpu.VMEM((IH,), jnp.int32) for _ in range(C // IH)],
            [pltpu.VMEM((IH,), jnp.int32) for _ in range(C // IH)],
            [pltpu.VMEM((IH,), jnp.int32) for _ in range(C // IH)],
            pltpu.VMEM((C, DIM), jnp.float32),
            pltpu.VMEM((C, DIM), jnp.float32),
            pltpu.VMEM((C, DIM), jnp.float32),
            pltpu.VMEM((C, DIM), jnp.float32),
            pltpu.VMEM((DIM, L), jnp.float32),
            pltpu.VMEM((PW,), jnp.float32),
            pltpu.SemaphoreType.DMA,
        ],
    )
    def _k(h_hbm, r_hbm, t_hbm, ent_hbm, rel_hbm, nrm_hbm, out_hbm,
           hidx, tidx, ridx, hrows, trows, rrows, nrows, u_scr, outv, sem):
        wid = lax.axis_index("s") * NC + lax.axis_index("c")
        lane = lax.iota(jnp.int32, L)
        last_lane = lane == (L - 1)

        for ch in range(NCH):
            base = wid * PW + ch * C
            for i in range(C // IH):
                pltpu.sync_copy(h_hbm.at[pl.ds(base + i * IH, IH)], hidx[i])
                pltpu.sync_copy(t_hbm.at[pl.ds(base + i * IH, IH)], tidx[i])
                pltpu.sync_copy(r_hbm.at[pl.ds(base + i * IH, IH)], ridx[i])
            copies = []
            for i in range(C // IH):
                s = i * IH
                copies.append(pltpu.async_copy(
                    ent_hbm.at[hidx[i]], hrows.at[pl.ds(s, IH)], sem))
                copies.append(pltpu.async_copy(
                    ent_hbm.at[tidx[i]], trows.at[pl.ds(s, IH)], sem))
                copies.append(pltpu.async_copy(
                    rel_hbm.at[ridx[i]], rrows.at[pl.ds(s, IH)], sem))
                copies.append(pltpu.async_copy(
                    nrm_hbm.at[ridx[i]], nrows.at[pl.ds(s, IH)], sem))
            for cp in copies:
                cp.wait()

            @pl.loop(0, C // L)
            def _group(g):
                row = g * L + lane
                cd = jnp.zeros((L,), jnp.int32)
                un = jnp.zeros((L,), jnp.float32)
                nn = jnp.zeros((L,), jnp.float32)
                for d in range(DIM):
                    hv = plsc.load_gather(hrows, [row, cd])
                    tv = plsc.load_gather(trows, [row, cd])
                    nv = plsc.load_gather(nrows, [row, cd])
                    uv = hv - tv
                    u_scr[d] = uv
                    un = un + uv * nv
                    nn = nn + nv * nv
                    if d + 1 < DIM:
                        cd = cd + 1
                gamma = un / jnp.maximum(nn, 1e-24)
                cd2 = jnp.zeros((L,), jnp.int32)
                acc = jnp.zeros((L,), jnp.float32)
                for d in range(DIM):
                    rv = plsc.load_gather(rrows, [row, cd2])
                    nv = plsc.load_gather(nrows, [row, cd2])
                    acc = acc + jnp.abs(u_scr[d] + rv - gamma * nv)
                    if d + 1 < DIM:
                        cd2 = cd2 + 1
                outv[pl.ds(ch * C + g * L, L)] = acc

        pltpu.sync_copy(outv, out_hbm.at[pl.ds(wid * PW, PW)])

    return _k(h_ids, r_ids, t_ids, entity_emb, relation_emb, normal_vec)


def kernel(h_ids, r_ids, t_ids, entity_emb, relation_emb, normal_vec):
    return _transh_sc(h_ids, r_ids, t_ids, entity_emb, relation_emb, normal_vec)
